# baseline (speedup 1.0000x reference)
import functools

import jax
import jax.numpy as jnp
from jax import lax
from jax.experimental import pallas as pl
from jax.experimental.pallas import tpu as pltpu
from jax.experimental.pallas import tpu_sc as plsc

VOCAB = 100000
EMB = 128
BATCH = 4096
SEQ = 200

NTOT = BATCH * SEQ
NW = 32
PER_W = NTOT // NW
CHUNK = 128
NCH = PER_W // CHUNK
NBUF = 5
LA = 2

assert NCH % NBUF == 0


@functools.cache
def _build_kernel():
    mesh = plsc.VectorSubcoreMesh(core_axis_name="c", subcore_axis_name="s")
    return functools.partial(
        pl.kernel,
        mesh=mesh,
        out_type=jax.ShapeDtypeStruct((NTOT, EMB), jnp.float32),
        scratch_types=[
            pltpu.VMEM((NCH, CHUNK), jnp.int32),
            pltpu.VMEM((NBUF, CHUNK, EMB), jnp.float32),
            pltpu.SemaphoreType.DMA((NBUF,)),
            pltpu.SemaphoreType.DMA((NBUF,)),
        ],
    )(_embed_body)


def _embed_body(x_hbm, tab_hbm, out_hbm, idx_v, rows_v, gsem, ssem):
    wid = lax.axis_index("s") * 2 + lax.axis_index("c")
    base = wid * PER_W

    pltpu.sync_copy(x_hbm.at[wid], idx_v)

    def gather_start(c, b):
        pltpu.async_copy(
            tab_hbm.at[idx_v.at[c]], rows_v.at[b], gsem.at[b]
        )

    def gather_wait(c, b):
        pltpu.make_async_copy(
            tab_hbm.at[idx_v.at[c]], rows_v.at[b], gsem.at[b]
        ).wait()

    def scatter_start(c, b):
        pltpu.async_copy(
            rows_v.at[b], out_hbm.at[pl.ds(base + c * CHUNK, CHUNK)],
            ssem.at[b],
        )

    def scatter_wait(c, b):
        pltpu.make_async_copy(
            rows_v.at[b], out_hbm.at[pl.ds(base + c * CHUNK, CHUNK)],
            ssem.at[b],
        ).wait()

    for c in range(LA):
        gather_start(c, c)

    def body(i, _):
        for b0 in range(NBUF):
            c = i * NBUF + b0
            gather_wait(c, b0)
            scatter_start(c, b0)
            b2 = (b0 + LA) % NBUF

            @pl.when(c + LA - NBUF >= 0)
            def _():
                scatter_wait(c + LA - NBUF, b2)

            @pl.when(c + LA < NCH)
            def _():
                gather_start(c + LA, b2)

        return 0

    lax.fori_loop(0, NCH // NBUF, body, 0)

    for c in range(NCH - (NBUF - LA), NCH):
        scatter_wait(c, c % NBUF)


def kernel(x, embed_weight):
    x3 = x.reshape(NW, NCH, CHUNK)
    out = _build_kernel()(x3, embed_weight)
    return out.reshape(BATCH, SEQ, EMB)


# speedup vs baseline: 1.0049x; 1.0049x over previous
import functools

import jax
import jax.numpy as jnp
from jax import lax
from jax.experimental import pallas as pl
from jax.experimental.pallas import tpu as pltpu
from jax.experimental.pallas import tpu_sc as plsc

VOCAB = 100000
EMB = 128
BATCH = 4096
SEQ = 200

NTOT = BATCH * SEQ
NW = 32
PER_W = NTOT // NW
CHUNK = 128
NCH = PER_W // CHUNK
NBUF = 5
LA = 3

assert NCH % NBUF == 0


@functools.cache
def _build_kernel():
    mesh = plsc.VectorSubcoreMesh(core_axis_name="c", subcore_axis_name="s")
    return functools.partial(
        pl.kernel,
        mesh=mesh,
        out_type=jax.ShapeDtypeStruct((NTOT, EMB), jnp.float32),
        scratch_types=[
            pltpu.VMEM((NCH, CHUNK), jnp.int32),
            pltpu.VMEM((NBUF, CHUNK, EMB), jnp.float32),
            pltpu.SemaphoreType.DMA((NBUF,)),
            pltpu.SemaphoreType.DMA((NBUF,)),
        ],
    )(_embed_body)


def _embed_body(x_hbm, tab_hbm, out_hbm, idx_v, rows_v, gsem, ssem):
    wid = lax.axis_index("s") * 2 + lax.axis_index("c")
    base = wid * PER_W

    pltpu.sync_copy(x_hbm.at[wid], idx_v)

    def gather_start(c, b):
        pltpu.async_copy(
            tab_hbm.at[idx_v.at[c]], rows_v.at[b], gsem.at[b]
        )

    def gather_wait(c, b):
        pltpu.make_async_copy(
            tab_hbm.at[idx_v.at[c]], rows_v.at[b], gsem.at[b]
        ).wait()

    def scatter_start(c, b):
        pltpu.async_copy(
            rows_v.at[b], out_hbm.at[pl.ds(base + c * CHUNK, CHUNK)],
            ssem.at[b],
        )

    def scatter_wait(c, b):
        pltpu.make_async_copy(
            rows_v.at[b], out_hbm.at[pl.ds(base + c * CHUNK, CHUNK)],
            ssem.at[b],
        ).wait()

    for c in range(LA):
        gather_start(c, c)

    def body(i, _):
        for b0 in range(NBUF):
            c = i * NBUF + b0
            gather_wait(c, b0)
            scatter_start(c, b0)
            b2 = (b0 + LA) % NBUF

            @pl.when(c + LA - NBUF >= 0)
            def _():
                scatter_wait(c + LA - NBUF, b2)

            @pl.when(c + LA < NCH)
            def _():
                gather_start(c + LA, b2)

        return 0

    lax.fori_loop(0, NCH // NBUF, body, 0)

    for c in range(NCH - (NBUF - LA), NCH):
        scatter_wait(c, c % NBUF)


def kernel(x, embed_weight):
    x3 = x.reshape(NW, NCH, CHUNK)
    out = _build_kernel()(x3, embed_weight)
    return out.reshape(BATCH, SEQ, EMB)


# speedup vs baseline: 1.0382x; 1.0331x over previous
import functools

import jax
import jax.numpy as jnp
from jax import lax
from jax.experimental import pallas as pl
from jax.experimental.pallas import tpu as pltpu
from jax.experimental.pallas import tpu_sc as plsc

VOCAB = 100000
EMB = 128
BATCH = 4096
SEQ = 200

NTOT = BATCH * SEQ
NW = 32
PER_W = NTOT // NW
CHUNK = 128
NCH = PER_W // CHUNK
NBUF = 5
LA = 3
SP_SLOTS = (1, 3)

assert NCH % NBUF == 0


@functools.cache
def _build_kernel():
    mesh = plsc.VectorSubcoreMesh(core_axis_name="c", subcore_axis_name="s")
    return functools.partial(
        pl.kernel,
        mesh=mesh,
        out_type=jax.ShapeDtypeStruct((NTOT, EMB), jnp.float32),
        scratch_types=[
            pltpu.VMEM((NCH, CHUNK), jnp.int32),
            pltpu.VMEM((NBUF, CHUNK, EMB), jnp.float32),
            pltpu.VMEM_SHARED((16, CHUNK, EMB), jnp.float32),
            pltpu.SemaphoreType.DMA((NBUF,)),
            pltpu.SemaphoreType.DMA((NBUF,)),
            pltpu.SemaphoreType.DMA,
        ],
    )(_embed_body)


def _embed_body(x_hbm, tab_hbm, out_hbm, idx_v, rows_v, sp_stage, gsem, ssem,
                sp_sem):
    sid = lax.axis_index("s")
    wid = sid * 2 + lax.axis_index("c")
    base = wid * PER_W

    pltpu.sync_copy(x_hbm.at[wid], idx_v)

    def gather_start(c, b):
        pltpu.async_copy(
            tab_hbm.at[idx_v.at[c]], rows_v.at[b], gsem.at[b]
        )

    def gather_wait(c, b):
        pltpu.make_async_copy(
            tab_hbm.at[idx_v.at[c]], rows_v.at[b], gsem.at[b]
        ).wait()

    def out_slice(c):
        return out_hbm.at[pl.ds(base + c * CHUNK, CHUNK)]

    def scatter_start(c, b):
        pltpu.async_copy(rows_v.at[b], out_slice(c), ssem.at[b])

    def scatter_wait(c, b):
        pltpu.make_async_copy(rows_v.at[b], out_slice(c), ssem.at[b]).wait()

    def sp_wait(c):
        pltpu.make_async_copy(sp_stage.at[sid], out_slice(c), sp_sem).wait()

    def sp_write(c):
        @pl.when(c >= 2)
        def _():
            sp_wait(c)

        pltpu.sync_copy(rows_v.at[c % NBUF], sp_stage.at[sid])
        pltpu.async_copy(sp_stage.at[sid], out_slice(c), sp_sem)

    for c in range(LA):
        gather_start(c, c)

    def body(i, _):
        for b0 in range(NBUF):
            c = i * NBUF + b0
            gather_wait(c, b0)
            if b0 in SP_SLOTS:
                sp_write(c)
            else:
                scatter_start(c, b0)
            b2 = (b0 + LA) % NBUF

            if b2 not in SP_SLOTS:
                @pl.when(c + LA - NBUF >= 0)
                def _():
                    scatter_wait(c + LA - NBUF, b2)

            @pl.when(c + LA < NCH)
            def _():
                gather_start(c + LA, b2)

        return 0

    lax.fori_loop(0, NCH // NBUF, body, 0)

    for c in range(NCH - (NBUF - LA), NCH):
        if (c % NBUF) not in SP_SLOTS:
            scatter_wait(c, c % NBUF)
    last_sp = max(c for c in range(NCH) if (c % NBUF) in SP_SLOTS)
    sp_wait(last_sp)


def kernel(x, embed_weight):
    x3 = x.reshape(NW, NCH, CHUNK)
    out = _build_kernel()(x3, embed_weight)
    return out.reshape(BATCH, SEQ, EMB)
